# baseline (speedup 1.0000x reference)
import jax
import jax.numpy as jnp
from jax import lax
from jax.experimental import pallas as pl
from jax.experimental.pallas import tpu as pltpu
from jax.experimental.pallas import tpu_sc as plsc

B, C, NPTS, K, NTOT = 4, 64, 4096, 32, 8192
CG = 4
NCHUNK = 128
KC = 16
GRPS = NCHUNK // 16
NCK = 2 * (NPTS // NCHUNK)


def _shuf(v, idx):
    dnums = lax.GatherDimensionNumbers(
        offset_dims=(), collapsed_slice_dims=(0,), start_index_map=(0,))
    return lax.gather(v, idx[:, None], dnums, (1,),
                      mode=lax.GatherScatterMode.PROMISE_IN_BOUNDS)


def _sc_body(xt, gif, fps, out, dil, vbuf0, vbuf1, gbuf0, gbuf1, probe,
             fbuf, obuf, vsem0, vsem1, gsem0, gsem1):
    cid = lax.axis_index("c")
    sid = lax.axis_index("s")
    b = cid
    c0 = sid * CG
    lanes = lax.iota(jnp.int32, 16)
    vbufs = (vbuf0, vbuf1)
    gbufs = (gbuf0, gbuf1)
    vsems = (vsem0, vsem1)
    gsems = (gsem0, gsem1)

    def start(ci, par):
        k0 = (ci % 2) * KC
        n0 = (ci // 2) * NCHUNK
        pltpu.async_copy(gif.at[b, pl.ds(k0, KC), pl.ds(n0, NCHUNK)],
                         gbufs[par], gsems[par])
        pltpu.async_copy(
            xt.at[b, pl.ds(c0, CG), pl.ds(k0, KC), pl.ds(n0, NCHUNK)],
            vbufs[par], vsems[par])

    def wait(par):
        pltpu.make_async_copy(gif.at[0, pl.ds(0, KC), pl.ds(0, NCHUNK)],
                              gbufs[par], gsems[par]).wait()
        pltpu.make_async_copy(
            xt.at[0, pl.ds(0, CG), pl.ds(0, KC), pl.ds(0, NCHUNK)],
            vbufs[par], vsems[par]).wait()

    def zero_body(i, _):
        for u in range(4):
            dil[pl.ds(i * 64 + u * 16, 16)] = jnp.zeros((16,), jnp.float32)
        return 0
    lax.fori_loop(0, NTOT * CG // 64, zero_body, 0)

    start(0, 0)
    start(1, 1)

    def chunk_body(half, _):
        for par in range(2):
            ci = half * 2 + par
            wait(par)
            vbuf = vbufs[par]
            gbuf = gbufs[par]

            def k_body(k, _):
                def j_body(jj, _):
                  for u in range(2):
                    j = jj * 2 + u
                    iv = gbuf[k, pl.ds(j * 16, 16)]
                    plsc.store_scatter(probe, [iv], lanes)
                    rb = plsc.load_gather(probe, [iv])
                    dup = jnp.any(rb != lanes)
                    base = iv * CG

                    @pl.when(jnp.logical_not(dup))
                    def _fast():
                        vals = [vbuf[ch, k, pl.ds(j * 16, 16)]
                                for ch in range(CG)]
                        curs = [plsc.load_gather(dil, [base + ch])
                                for ch in range(CG)]
                        for ch in range(CG):
                            plsc.store_scatter(
                                dil, [base + ch],
                                jnp.maximum(curs[ch], vals[ch]))

                    @pl.when(dup)
                    def _slow():
                        key = iv * 16 + lanes
                        sk, perm = plsc.sort_key_val(key, lanes)
                        iv_s = sk // 16
                        base_s = iv_s * CG
                        folds = []
                        for d in (1, 2, 4, 8):
                            si = jnp.maximum(lanes - d, 0)
                            folds.append((_shuf(iv_s, si) == iv_s, si))
                        nxt = jnp.minimum(lanes + 1, 15)
                        writer = jnp.logical_or(_shuf(iv_s, nxt) != iv_s,
                                                lanes == 15)
                        for ch in range(CG):
                            vals = vbuf[ch, k, pl.ds(j * 16, 16)]
                            v_s = _shuf(vals, perm)
                            for eq, si in folds:
                                v_s = jnp.where(
                                    eq, jnp.maximum(v_s, _shuf(v_s, si)), v_s)
                            addr = base_s + ch
                            cur = plsc.load_gather(dil, [addr])
                            plsc.store_scatter(dil, [addr],
                                               jnp.maximum(cur, v_s),
                                               mask=writer)
                  return 0

                lax.fori_loop(0, GRPS // 2, j_body, 0)
                return 0

            lax.fori_loop(0, KC, k_body, 0)

            @pl.when(ci + 2 < NCK)
            def _prefetch():
                start(ci + 2, par)
        return 0

    lax.fori_loop(0, NCK // 2, chunk_body, 0)

    def gchunk_body(ci, _):
        n0 = ci * NCHUNK
        pltpu.sync_copy(fps.at[b, pl.ds(n0, NCHUNK)], fbuf)

        def ggrp(j, _):
            fv = fbuf[pl.ds(j * 16, 16)]
            gb = fv * CG
            gs = [plsc.load_gather(dil, [gb + ch]) for ch in range(CG)]
            for ch in range(CG):
                obuf[ch, pl.ds(j * 16, 16)] = gs[ch]
            return 0

        lax.fori_loop(0, GRPS, ggrp, 0)
        pltpu.sync_copy(obuf, out.at[b, pl.ds(c0, CG), pl.ds(n0, NCHUNK)])
        return 0

    lax.fori_loop(0, NPTS // NCHUNK, gchunk_body, 0)


@jax.jit
def _sc_scatter_gather(xt, gif, fps):
    mesh = plsc.VectorSubcoreMesh(core_axis_name="c", subcore_axis_name="s")
    return pl.kernel(
        _sc_body,
        mesh=mesh,
        compiler_params=pltpu.CompilerParams(needs_layout_passes=False),
        out_type=jax.ShapeDtypeStruct((2, C, NPTS), jnp.float32),
        scratch_types=[
            pltpu.VMEM((NTOT * CG,), jnp.float32),
            pltpu.VMEM((CG, KC, NCHUNK), jnp.float32),
            pltpu.VMEM((CG, KC, NCHUNK), jnp.float32),
            pltpu.VMEM((KC, NCHUNK), jnp.int32),
            pltpu.VMEM((KC, NCHUNK), jnp.int32),
            pltpu.VMEM((NTOT,), jnp.int32),
            pltpu.VMEM((NCHUNK,), jnp.int32),
            pltpu.VMEM((CG, NCHUNK), jnp.float32),
            pltpu.SemaphoreType.DMA,
            pltpu.SemaphoreType.DMA,
            pltpu.SemaphoreType.DMA,
            pltpu.SemaphoreType.DMA,
        ],
    )(xt, gif, fps)


def _gx_kernel(xt_ref, o_ref):
    o_ref[...] = jnp.max(xt_ref[...], axis=2)


def _mlp_kernel(gx0_ref, gx1_ref, dx0_ref, dx1_ref, w1a_ref, w1b_ref,
                b1_ref, g1_ref, be1_ref, w2_ref, b2_ref, g2_ref, be2_ref,
                o_ref):
    eps = 1e-5
    nrm = 1.0 / (B * NPTS)

    w1a = w1a_ref[...]
    w1b = w1b_ref[...]
    gxs = [gx0_ref[0], gx0_ref[1], gx1_ref[0], gx1_ref[1]]
    dxs = [dx0_ref[0], dx0_ref[1], dx1_ref[0], dx1_ref[1]]
    h1 = [jnp.dot(w1a, gxs[b]) + jnp.dot(w1b, dxs[b])
          + b1_ref[...][:, None] for b in range(B)]
    m1 = sum(jnp.sum(h, axis=1) for h in h1) * nrm
    v1 = sum(jnp.sum((h - m1[:, None]) ** 2, axis=1) for h in h1) * nrm
    s1 = g1_ref[...] / jnp.sqrt(v1 + eps)
    r1 = [jnp.maximum((h - m1[:, None]) * s1[:, None] + be1_ref[...][:, None],
                      0.0) for h in h1]

    w2 = w2_ref[...]
    h2 = [jnp.dot(w2, r) + b2_ref[...][:, None] for r in r1]
    m2 = sum(jnp.sum(h, axis=1) for h in h2) * nrm
    v2 = sum(jnp.sum((h - m2[:, None]) ** 2, axis=1) for h in h2) * nrm
    s2 = g2_ref[...] / jnp.sqrt(v2 + eps)
    for b in range(B):
        o_ref[b] = jnp.maximum(
            (h2[b] - m2[:, None]) * s2[:, None] + be2_ref[...][:, None], 0.0)


def _gx_call(xt):
    return pl.pallas_call(
        _gx_kernel,
        grid=(2, C // 8, NPTS // 512),
        in_specs=[pl.BlockSpec((1, 8, K, 512), lambda i, j, l: (i, j, 0, l))],
        out_specs=pl.BlockSpec((1, 8, 512), lambda i, j, l: (i, j, l)),
        out_shape=jax.ShapeDtypeStruct((2, C, NPTS), jnp.float32),
    )(xt)


def kernel(x, group_idx, fps_idx, N, W1, b1, gamma1, beta1, W2, b2, gamma2,
           beta2):
    xt0 = jnp.transpose(x[:2], (0, 1, 3, 2))
    gif0 = jnp.transpose(group_idx[:2], (0, 2, 1))
    dil0 = _sc_scatter_gather(xt0, gif0, fps_idx[:2])
    xt1 = jnp.transpose(x[2:], (0, 1, 3, 2))
    gif1 = jnp.transpose(group_idx[2:], (0, 2, 1))
    dil1 = _sc_scatter_gather(xt1, gif1, fps_idx[2:])
    gx0 = _gx_call(xt0)
    gx1 = _gx_call(xt1)

    out = pl.pallas_call(
        _mlp_kernel,
        out_shape=jax.ShapeDtypeStruct((B, C, NPTS), jnp.float32),
    )(gx0, gx1, dil0, dil1, W1[:, :C], W1[:, C:], b1, gamma1, beta1, W2, b2,
      gamma2, beta2)
    return out


# speedup vs baseline: 1.8514x; 1.8514x over previous
import jax
import jax.numpy as jnp
from jax import lax
from jax.experimental import pallas as pl
from jax.experimental.pallas import tpu as pltpu
from jax.experimental.pallas import tpu_sc as plsc

B, C, NPTS, K, NTOT = 4, 64, 4096, 32, 8192
CG = 8
NCHUNK = 128
KC = 16
GRPS = NCHUNK // 16
NCK = 2 * (NPTS // NCHUNK)


def _shuf(v, idx):
    dnums = lax.GatherDimensionNumbers(
        offset_dims=(), collapsed_slice_dims=(0,), start_index_map=(0,))
    return lax.gather(v, idx[:, None], dnums, (1,),
                      mode=lax.GatherScatterMode.PROMISE_IN_BOUNDS)


def _sc_body(xt, gif, fps, dilx, gxo, dil, vbuf0, vbuf1, gbuf0, gbuf1, probe,
             fbuf, obuf, gxb, vsem0, vsem1, gsem0, gsem1):
    cid = lax.axis_index("c")
    sid = lax.axis_index("s")
    b = cid * 2 + sid // 8
    cg = sid % 8
    c0 = cg * CG
    lanes = lax.iota(jnp.int32, 16)
    vbufs = (vbuf0, vbuf1)
    gbufs = (gbuf0, gbuf1)
    vsems = (vsem0, vsem1)
    gsems = (gsem0, gsem1)
    neg_inf = jnp.full((16,), -jnp.inf, jnp.float32)

    def start(ci, par):
        k0 = (ci % 2) * KC
        n0 = (ci // 2) * NCHUNK
        pltpu.async_copy(gif.at[b, pl.ds(k0, KC), pl.ds(n0, NCHUNK)],
                         gbufs[par], gsems[par])
        pltpu.async_copy(
            xt.at[b, pl.ds(c0, CG), pl.ds(k0, KC), pl.ds(n0, NCHUNK)],
            vbufs[par], vsems[par])

    def wait(par):
        pltpu.make_async_copy(gif.at[0, pl.ds(0, KC), pl.ds(0, NCHUNK)],
                              gbufs[par], gsems[par]).wait()
        pltpu.make_async_copy(
            xt.at[0, pl.ds(0, CG), pl.ds(0, KC), pl.ds(0, NCHUNK)],
            vbufs[par], vsems[par]).wait()

    def zero_body(i, _):
        for u in range(4):
            dil[pl.ds(i * 64 + u * 16, 16)] = jnp.zeros((16,), jnp.float32)
        return 0
    lax.fori_loop(0, NTOT * CG // 64, zero_body, 0)

    start(0, 0)
    start(1, 1)

    def chunk_body(half, _):
        for par in range(2):
            ci = half * 2 + par
            kh = ci % 2
            n0 = (ci // 2) * NCHUNK
            wait(par)
            vbuf = vbufs[par]
            gbuf = gbufs[par]

            def j_body(j, _):
                def k_body(k, gxc):
                    iv = gbuf[k, pl.ds(j * 16, 16)]
                    vals = [vbuf[ch, k, pl.ds(j * 16, 16)]
                            for ch in range(CG)]
                    plsc.store_scatter(probe, [iv], lanes)
                    rb = plsc.load_gather(probe, [iv])
                    dup = jnp.any(rb != lanes)
                    base = iv * CG

                    @pl.when(jnp.logical_not(dup))
                    def _fast():
                        curs = [plsc.load_gather(dil, [base + ch])
                                for ch in range(CG)]
                        for ch in range(CG):
                            plsc.store_scatter(
                                dil, [base + ch],
                                jnp.maximum(curs[ch], vals[ch]))

                    @pl.when(dup)
                    def _slow():
                        key = iv * 16 + lanes
                        sk, perm = plsc.sort_key_val(key, lanes)
                        iv_s = sk // 16
                        base_s = iv_s * CG
                        folds = []
                        for d in (1, 2, 4, 8):
                            si = jnp.maximum(lanes - d, 0)
                            folds.append((_shuf(iv_s, si) == iv_s, si))
                        nxt = jnp.minimum(lanes + 1, 15)
                        writer = jnp.logical_or(_shuf(iv_s, nxt) != iv_s,
                                                lanes == 15)
                        for ch in range(CG):
                            v_s = _shuf(vals[ch], perm)
                            for eq, si in folds:
                                v_s = jnp.where(
                                    eq, jnp.maximum(v_s, _shuf(v_s, si)), v_s)
                            addr = base_s + ch
                            cur = plsc.load_gather(dil, [addr])
                            plsc.store_scatter(dil, [addr],
                                               jnp.maximum(cur, v_s),
                                               mask=writer)

                    return tuple(jnp.maximum(gxc[ch], vals[ch])
                                 for ch in range(CG))

                gxc = lax.fori_loop(0, KC, k_body,
                                    tuple(neg_inf for _ in range(CG)))

                @pl.when(kh == 0)
                def _store_gx():
                    for ch in range(CG):
                        gxb[ch, pl.ds(j * 16, 16)] = gxc[ch]

                @pl.when(kh == 1)
                def _merge_gx():
                    for ch in range(CG):
                        gxb[ch, pl.ds(j * 16, 16)] = jnp.maximum(
                            gxb[ch, pl.ds(j * 16, 16)], gxc[ch])
                return 0

            lax.fori_loop(0, GRPS, j_body, 0)

            @pl.when(kh == 1)
            def _flush_gx():
                pltpu.sync_copy(gxb,
                                gxo.at[b, pl.ds(c0, CG), pl.ds(n0, NCHUNK)])

            @pl.when(ci + 2 < NCK)
            def _prefetch():
                start(ci + 2, par)
        return 0

    lax.fori_loop(0, NCK // 2, chunk_body, 0)

    def gchunk_body(ci, _):
        n0 = ci * NCHUNK
        pltpu.sync_copy(fps.at[b, pl.ds(n0, NCHUNK)], fbuf)

        def ggrp(j, _):
            fv = fbuf[pl.ds(j * 16, 16)]
            gb = fv * CG
            gs = [plsc.load_gather(dil, [gb + ch]) for ch in range(CG)]
            for ch in range(CG):
                obuf[ch, pl.ds(j * 16, 16)] = gs[ch]
            return 0

        lax.fori_loop(0, GRPS, ggrp, 0)
        pltpu.sync_copy(obuf, dilx.at[b, pl.ds(c0, CG), pl.ds(n0, NCHUNK)])
        return 0

    lax.fori_loop(0, NPTS // NCHUNK, gchunk_body, 0)


@jax.jit
def _sc_scatter_gather(xt, gif, fps):
    mesh = plsc.VectorSubcoreMesh(core_axis_name="c", subcore_axis_name="s")
    return pl.kernel(
        _sc_body,
        mesh=mesh,
        compiler_params=pltpu.CompilerParams(needs_layout_passes=False),
        out_type=[jax.ShapeDtypeStruct((B, C, NPTS), jnp.float32),
                  jax.ShapeDtypeStruct((B, C, NPTS), jnp.float32)],
        scratch_types=[
            pltpu.VMEM((NTOT * CG,), jnp.float32),
            pltpu.VMEM((CG, KC, NCHUNK), jnp.float32),
            pltpu.VMEM((CG, KC, NCHUNK), jnp.float32),
            pltpu.VMEM((KC, NCHUNK), jnp.int32),
            pltpu.VMEM((KC, NCHUNK), jnp.int32),
            pltpu.VMEM((NTOT,), jnp.int32),
            pltpu.VMEM((NCHUNK,), jnp.int32),
            pltpu.VMEM((CG, NCHUNK), jnp.float32),
            pltpu.VMEM((CG, NCHUNK), jnp.float32),
            pltpu.SemaphoreType.DMA,
            pltpu.SemaphoreType.DMA,
            pltpu.SemaphoreType.DMA,
            pltpu.SemaphoreType.DMA,
        ],
    )(xt, gif, fps)


def _mlp_kernel(gx_ref, dx_ref, w1a_ref, w1b_ref, b1_ref, g1_ref, be1_ref,
                w2_ref, b2_ref, g2_ref, be2_ref, o_ref):
    eps = 1e-5
    nrm = 1.0 / (B * NPTS)

    w1a = w1a_ref[...]
    w1b = w1b_ref[...]
    h1 = [jnp.dot(w1a, gx_ref[b]) + jnp.dot(w1b, dx_ref[b])
          + b1_ref[...][:, None] for b in range(B)]
    m1 = sum(jnp.sum(h, axis=1) for h in h1) * nrm
    v1 = sum(jnp.sum((h - m1[:, None]) ** 2, axis=1) for h in h1) * nrm
    s1 = g1_ref[...] / jnp.sqrt(v1 + eps)
    r1 = [jnp.maximum((h - m1[:, None]) * s1[:, None] + be1_ref[...][:, None],
                      0.0) for h in h1]

    w2 = w2_ref[...]
    h2 = [jnp.dot(w2, r) + b2_ref[...][:, None] for r in r1]
    m2 = sum(jnp.sum(h, axis=1) for h in h2) * nrm
    v2 = sum(jnp.sum((h - m2[:, None]) ** 2, axis=1) for h in h2) * nrm
    s2 = g2_ref[...] / jnp.sqrt(v2 + eps)
    for b in range(B):
        o_ref[b] = jnp.maximum(
            (h2[b] - m2[:, None]) * s2[:, None] + be2_ref[...][:, None], 0.0)


def kernel(x, group_idx, fps_idx, N, W1, b1, gamma1, beta1, W2, b2, gamma2,
           beta2):
    xt = jnp.transpose(x, (0, 1, 3, 2))
    gif = jnp.transpose(group_idx, (0, 2, 1))

    dil_x, gx = _sc_scatter_gather(xt, gif, fps_idx)

    out = pl.pallas_call(
        _mlp_kernel,
        out_shape=jax.ShapeDtypeStruct((B, C, NPTS), jnp.float32),
    )(gx, dil_x, W1[:, :C], W1[:, C:], b1, gamma1, beta1, W2, b2, gamma2,
      beta2)
    return out


# speedup vs baseline: 1.9297x; 1.0423x over previous
import jax
import jax.numpy as jnp
from jax import lax
from jax.experimental import pallas as pl
from jax.experimental.pallas import tpu as pltpu
from jax.experimental.pallas import tpu_sc as plsc

B, C, NPTS, K, NTOT = 4, 64, 4096, 32, 8192
CG = 8
NCHUNK = 128
KC = 16
GRPS = NCHUNK // 16
NCK = 2 * (NPTS // NCHUNK)


def _shuf(v, idx):
    dnums = lax.GatherDimensionNumbers(
        offset_dims=(), collapsed_slice_dims=(0,), start_index_map=(0,))
    return lax.gather(v, idx[:, None], dnums, (1,),
                      mode=lax.GatherScatterMode.PROMISE_IN_BOUNDS)


def _sc_body(xt, gif, fps, dilx, gxo, dil, vbuf0, vbuf1, gbuf0, gbuf1, probe,
             fbuf, obuf, gxb, vsem0, vsem1, gsem0, gsem1):
    cid = lax.axis_index("c")
    sid = lax.axis_index("s")
    b = cid * 2 + sid // 8
    cg = sid % 8
    c0 = cg * CG
    lanes = lax.iota(jnp.int32, 16)
    vbufs = (vbuf0, vbuf1)
    gbufs = (gbuf0, gbuf1)
    vsems = (vsem0, vsem1)
    gsems = (gsem0, gsem1)
    neg_inf = jnp.full((16,), -jnp.inf, jnp.float32)

    def start(ci, par):
        k0 = (ci % 2) * KC
        n0 = (ci // 2) * NCHUNK
        pltpu.async_copy(gif.at[b, pl.ds(k0, KC), pl.ds(n0, NCHUNK)],
                         gbufs[par], gsems[par])
        pltpu.async_copy(
            xt.at[b, pl.ds(c0, CG), pl.ds(k0, KC), pl.ds(n0, NCHUNK)],
            vbufs[par], vsems[par])

    def wait(par):
        pltpu.make_async_copy(gif.at[0, pl.ds(0, KC), pl.ds(0, NCHUNK)],
                              gbufs[par], gsems[par]).wait()
        pltpu.make_async_copy(
            xt.at[0, pl.ds(0, CG), pl.ds(0, KC), pl.ds(0, NCHUNK)],
            vbufs[par], vsems[par]).wait()

    def zero_body(i, _):
        for u in range(4):
            dil[pl.ds(i * 64 + u * 16, 16)] = jnp.zeros((16,), jnp.float32)
        return 0
    lax.fori_loop(0, NTOT * CG // 64, zero_body, 0)

    start(0, 0)
    start(1, 1)

    def chunk_body(half, _):
        for par in range(2):
            ci = half * 2 + par
            kh = ci % 2
            n0 = (ci // 2) * NCHUNK
            wait(par)
            vbuf = vbufs[par]
            gbuf = gbufs[par]

            def j_body(j, _):
                def k_body(kk, gxc):
                  for u in range(2):
                    k = kk * 2 + u
                    iv = gbuf[k, pl.ds(j * 16, 16)]
                    vals = [vbuf[ch, k, pl.ds(j * 16, 16)]
                            for ch in range(CG)]
                    plsc.store_scatter(probe, [iv], lanes)
                    rb = plsc.load_gather(probe, [iv])
                    dup = jnp.any(rb != lanes)
                    base = iv * CG

                    @pl.when(jnp.logical_not(dup))
                    def _fast():
                        curs = [plsc.load_gather(dil, [base + ch])
                                for ch in range(CG)]
                        for ch in range(CG):
                            plsc.store_scatter(
                                dil, [base + ch],
                                jnp.maximum(curs[ch], vals[ch]))

                    @pl.when(dup)
                    def _slow():
                        key = iv * 16 + lanes
                        sk, perm = plsc.sort_key_val(key, lanes)
                        iv_s = sk // 16
                        base_s = iv_s * CG
                        folds = []
                        for d in (1, 2, 4, 8):
                            si = jnp.maximum(lanes - d, 0)
                            folds.append((_shuf(iv_s, si) == iv_s, si))
                        nxt = jnp.minimum(lanes + 1, 15)
                        writer = jnp.logical_or(_shuf(iv_s, nxt) != iv_s,
                                                lanes == 15)
                        for ch in range(CG):
                            v_s = _shuf(vals[ch], perm)
                            for eq, si in folds:
                                v_s = jnp.where(
                                    eq, jnp.maximum(v_s, _shuf(v_s, si)), v_s)
                            addr = base_s + ch
                            cur = plsc.load_gather(dil, [addr])
                            plsc.store_scatter(dil, [addr],
                                               jnp.maximum(cur, v_s),
                                               mask=writer)

                    gxc = tuple(jnp.maximum(gxc[ch], vals[ch])
                                for ch in range(CG))
                  return gxc

                gxc = lax.fori_loop(0, KC // 2, k_body,
                                    tuple(neg_inf for _ in range(CG)))

                @pl.when(kh == 0)
                def _store_gx():
                    for ch in range(CG):
                        gxb[ch, pl.ds(j * 16, 16)] = gxc[ch]

                @pl.when(kh == 1)
                def _merge_gx():
                    for ch in range(CG):
                        gxb[ch, pl.ds(j * 16, 16)] = jnp.maximum(
                            gxb[ch, pl.ds(j * 16, 16)], gxc[ch])
                return 0

            lax.fori_loop(0, GRPS, j_body, 0)

            @pl.when(kh == 1)
            def _flush_gx():
                pltpu.sync_copy(gxb,
                                gxo.at[b, pl.ds(c0, CG), pl.ds(n0, NCHUNK)])

            @pl.when(ci + 2 < NCK)
            def _prefetch():
                start(ci + 2, par)
        return 0

    lax.fori_loop(0, NCK // 2, chunk_body, 0)

    def gchunk_body(ci, _):
        n0 = ci * NCHUNK
        pltpu.sync_copy(fps.at[b, pl.ds(n0, NCHUNK)], fbuf)

        def ggrp(j, _):
            fv = fbuf[pl.ds(j * 16, 16)]
            gb = fv * CG
            gs = [plsc.load_gather(dil, [gb + ch]) for ch in range(CG)]
            for ch in range(CG):
                obuf[ch, pl.ds(j * 16, 16)] = gs[ch]
            return 0

        lax.fori_loop(0, GRPS, ggrp, 0)
        pltpu.sync_copy(obuf, dilx.at[b, pl.ds(c0, CG), pl.ds(n0, NCHUNK)])
        return 0

    lax.fori_loop(0, NPTS // NCHUNK, gchunk_body, 0)


@jax.jit
def _sc_scatter_gather(xt, gif, fps):
    mesh = plsc.VectorSubcoreMesh(core_axis_name="c", subcore_axis_name="s")
    return pl.kernel(
        _sc_body,
        mesh=mesh,
        compiler_params=pltpu.CompilerParams(needs_layout_passes=False),
        out_type=[jax.ShapeDtypeStruct((B, C, NPTS), jnp.float32),
                  jax.ShapeDtypeStruct((B, C, NPTS), jnp.float32)],
        scratch_types=[
            pltpu.VMEM((NTOT * CG,), jnp.float32),
            pltpu.VMEM((CG, KC, NCHUNK), jnp.float32),
            pltpu.VMEM((CG, KC, NCHUNK), jnp.float32),
            pltpu.VMEM((KC, NCHUNK), jnp.int32),
            pltpu.VMEM((KC, NCHUNK), jnp.int32),
            pltpu.VMEM((NTOT,), jnp.int32),
            pltpu.VMEM((NCHUNK,), jnp.int32),
            pltpu.VMEM((CG, NCHUNK), jnp.float32),
            pltpu.VMEM((CG, NCHUNK), jnp.float32),
            pltpu.SemaphoreType.DMA,
            pltpu.SemaphoreType.DMA,
            pltpu.SemaphoreType.DMA,
            pltpu.SemaphoreType.DMA,
        ],
    )(xt, gif, fps)


def _mlp_kernel(gx_ref, dx_ref, w1a_ref, w1b_ref, b1_ref, g1_ref, be1_ref,
                w2_ref, b2_ref, g2_ref, be2_ref, o_ref):
    eps = 1e-5
    nrm = 1.0 / (B * NPTS)

    w1a = w1a_ref[...]
    w1b = w1b_ref[...]
    h1 = [jnp.dot(w1a, gx_ref[b]) + jnp.dot(w1b, dx_ref[b])
          + b1_ref[...][:, None] for b in range(B)]
    m1 = sum(jnp.sum(h, axis=1) for h in h1) * nrm
    v1 = sum(jnp.sum((h - m1[:, None]) ** 2, axis=1) for h in h1) * nrm
    s1 = g1_ref[...] / jnp.sqrt(v1 + eps)
    r1 = [jnp.maximum((h - m1[:, None]) * s1[:, None] + be1_ref[...][:, None],
                      0.0) for h in h1]

    w2 = w2_ref[...]
    h2 = [jnp.dot(w2, r) + b2_ref[...][:, None] for r in r1]
    m2 = sum(jnp.sum(h, axis=1) for h in h2) * nrm
    v2 = sum(jnp.sum((h - m2[:, None]) ** 2, axis=1) for h in h2) * nrm
    s2 = g2_ref[...] / jnp.sqrt(v2 + eps)
    for b in range(B):
        o_ref[b] = jnp.maximum(
            (h2[b] - m2[:, None]) * s2[:, None] + be2_ref[...][:, None], 0.0)


def kernel(x, group_idx, fps_idx, N, W1, b1, gamma1, beta1, W2, b2, gamma2,
           beta2):
    xt = jnp.transpose(x, (0, 1, 3, 2))
    gif = jnp.transpose(group_idx, (0, 2, 1))

    dil_x, gx = _sc_scatter_gather(xt, gif, fps_idx)

    out = pl.pallas_call(
        _mlp_kernel,
        out_shape=jax.ShapeDtypeStruct((B, C, NPTS), jnp.float32),
    )(gx, dil_x, W1[:, :C], W1[:, C:], b1, gamma1, beta1, W2, b2, gamma2,
      beta2)
    return out


# speedup vs baseline: 1.9555x; 1.0134x over previous
import jax
import jax.numpy as jnp
from jax import lax
from jax.experimental import pallas as pl
from jax.experimental.pallas import tpu as pltpu
from jax.experimental.pallas import tpu_sc as plsc

B, C, NPTS, K, NTOT = 4, 64, 4096, 32, 8192
CG = 8
NCHUNK = 128
KC = 16
GRPS = NCHUNK // 16
NCK = 2 * (NPTS // NCHUNK)


def _shuf(v, idx):
    dnums = lax.GatherDimensionNumbers(
        offset_dims=(), collapsed_slice_dims=(0,), start_index_map=(0,))
    return lax.gather(v, idx[:, None], dnums, (1,),
                      mode=lax.GatherScatterMode.PROMISE_IN_BOUNDS)


def _sc_body(xt, gif, fps, dilx, gxo, dil, vbuf0, vbuf1, gbuf0, gbuf1, probe,
             fbuf, obuf, gxb, vsem0, vsem1, gsem0, gsem1):
    cid = lax.axis_index("c")
    sid = lax.axis_index("s")
    b = cid * 2 + sid // 8
    cg = sid % 8
    c0 = cg * CG
    lanes = lax.iota(jnp.int32, 16)
    vbufs = (vbuf0, vbuf1)
    gbufs = (gbuf0, gbuf1)
    vsems = (vsem0, vsem1)
    gsems = (gsem0, gsem1)
    neg_inf = jnp.full((16,), -jnp.inf, jnp.float32)

    def start(ci, par):
        k0 = (ci % 2) * KC
        n0 = (ci // 2) * NCHUNK
        pltpu.async_copy(gif.at[b, pl.ds(k0, KC), pl.ds(n0, NCHUNK)],
                         gbufs[par], gsems[par])
        pltpu.async_copy(
            xt.at[b, pl.ds(c0, CG), pl.ds(k0, KC), pl.ds(n0, NCHUNK)],
            vbufs[par], vsems[par])

    def wait(par):
        pltpu.make_async_copy(gif.at[0, pl.ds(0, KC), pl.ds(0, NCHUNK)],
                              gbufs[par], gsems[par]).wait()
        pltpu.make_async_copy(
            xt.at[0, pl.ds(0, CG), pl.ds(0, KC), pl.ds(0, NCHUNK)],
            vbufs[par], vsems[par]).wait()

    def zero_body(i, _):
        for u in range(4):
            dil[pl.ds(i * 64 + u * 16, 16)] = jnp.zeros((16,), jnp.float32)
        return 0
    lax.fori_loop(0, NTOT * CG // 64, zero_body, 0)

    start(0, 0)
    start(1, 1)

    def chunk_body(half, _):
        for par in range(2):
            ci = half * 2 + par
            kh = ci % 2
            n0 = (ci // 2) * NCHUNK
            wait(par)
            vbuf = vbufs[par]
            gbuf = gbufs[par]

            def j_body(j, _):
                def k_body(kk, gxc):
                  for u in range(4):
                    k = kk * 4 + u
                    iv = gbuf[k, pl.ds(j * 16, 16)]
                    vals = [vbuf[ch, k, pl.ds(j * 16, 16)]
                            for ch in range(CG)]
                    plsc.store_scatter(probe, [iv], lanes)
                    rb = plsc.load_gather(probe, [iv])
                    dup = jnp.any(rb != lanes)
                    base = iv * CG

                    @pl.when(jnp.logical_not(dup))
                    def _fast():
                        curs = [plsc.load_gather(dil, [base + ch])
                                for ch in range(CG)]
                        for ch in range(CG):
                            plsc.store_scatter(
                                dil, [base + ch],
                                jnp.maximum(curs[ch], vals[ch]))

                    @pl.when(dup)
                    def _slow():
                        key = iv * 16 + lanes
                        sk, perm = plsc.sort_key_val(key, lanes)
                        iv_s = sk // 16
                        base_s = iv_s * CG
                        folds = []
                        for d in (1, 2, 4, 8):
                            si = jnp.maximum(lanes - d, 0)
                            folds.append((_shuf(iv_s, si) == iv_s, si))
                        nxt = jnp.minimum(lanes + 1, 15)
                        writer = jnp.logical_or(_shuf(iv_s, nxt) != iv_s,
                                                lanes == 15)
                        for ch in range(CG):
                            v_s = _shuf(vals[ch], perm)
                            for eq, si in folds:
                                v_s = jnp.where(
                                    eq, jnp.maximum(v_s, _shuf(v_s, si)), v_s)
                            addr = base_s + ch
                            cur = plsc.load_gather(dil, [addr])
                            plsc.store_scatter(dil, [addr],
                                               jnp.maximum(cur, v_s),
                                               mask=writer)

                    gxc = tuple(jnp.maximum(gxc[ch], vals[ch])
                                for ch in range(CG))
                  return gxc

                gxc = lax.fori_loop(0, KC // 4, k_body,
                                    tuple(neg_inf for _ in range(CG)))

                @pl.when(kh == 0)
                def _store_gx():
                    for ch in range(CG):
                        gxb[ch, pl.ds(j * 16, 16)] = gxc[ch]

                @pl.when(kh == 1)
                def _merge_gx():
                    for ch in range(CG):
                        gxb[ch, pl.ds(j * 16, 16)] = jnp.maximum(
                            gxb[ch, pl.ds(j * 16, 16)], gxc[ch])
                return 0

            lax.fori_loop(0, GRPS, j_body, 0)

            @pl.when(kh == 1)
            def _flush_gx():
                pltpu.sync_copy(gxb,
                                gxo.at[b, pl.ds(c0, CG), pl.ds(n0, NCHUNK)])

            @pl.when(ci + 2 < NCK)
            def _prefetch():
                start(ci + 2, par)
        return 0

    lax.fori_loop(0, NCK // 2, chunk_body, 0)

    def gchunk_body(ci, _):
        n0 = ci * NCHUNK
        pltpu.sync_copy(fps.at[b, pl.ds(n0, NCHUNK)], fbuf)

        def ggrp(j, _):
            fv = fbuf[pl.ds(j * 16, 16)]
            gb = fv * CG
            gs = [plsc.load_gather(dil, [gb + ch]) for ch in range(CG)]
            for ch in range(CG):
                obuf[ch, pl.ds(j * 16, 16)] = gs[ch]
            return 0

        lax.fori_loop(0, GRPS, ggrp, 0)
        pltpu.sync_copy(obuf, dilx.at[b, pl.ds(c0, CG), pl.ds(n0, NCHUNK)])
        return 0

    lax.fori_loop(0, NPTS // NCHUNK, gchunk_body, 0)


@jax.jit
def _sc_scatter_gather(xt, gif, fps):
    mesh = plsc.VectorSubcoreMesh(core_axis_name="c", subcore_axis_name="s")
    return pl.kernel(
        _sc_body,
        mesh=mesh,
        compiler_params=pltpu.CompilerParams(needs_layout_passes=False),
        out_type=[jax.ShapeDtypeStruct((B, C, NPTS), jnp.float32),
                  jax.ShapeDtypeStruct((B, C, NPTS), jnp.float32)],
        scratch_types=[
            pltpu.VMEM((NTOT * CG,), jnp.float32),
            pltpu.VMEM((CG, KC, NCHUNK), jnp.float32),
            pltpu.VMEM((CG, KC, NCHUNK), jnp.float32),
            pltpu.VMEM((KC, NCHUNK), jnp.int32),
            pltpu.VMEM((KC, NCHUNK), jnp.int32),
            pltpu.VMEM((NTOT,), jnp.int32),
            pltpu.VMEM((NCHUNK,), jnp.int32),
            pltpu.VMEM((CG, NCHUNK), jnp.float32),
            pltpu.VMEM((CG, NCHUNK), jnp.float32),
            pltpu.SemaphoreType.DMA,
            pltpu.SemaphoreType.DMA,
            pltpu.SemaphoreType.DMA,
            pltpu.SemaphoreType.DMA,
        ],
    )(xt, gif, fps)


def _mlp_kernel(gx_ref, dx_ref, w1a_ref, w1b_ref, b1_ref, g1_ref, be1_ref,
                w2_ref, b2_ref, g2_ref, be2_ref, o_ref):
    eps = 1e-5
    nrm = 1.0 / (B * NPTS)

    w1a = w1a_ref[...]
    w1b = w1b_ref[...]
    h1 = [jnp.dot(w1a, gx_ref[b]) + jnp.dot(w1b, dx_ref[b])
          + b1_ref[...][:, None] for b in range(B)]
    m1 = sum(jnp.sum(h, axis=1) for h in h1) * nrm
    v1 = sum(jnp.sum((h - m1[:, None]) ** 2, axis=1) for h in h1) * nrm
    s1 = g1_ref[...] / jnp.sqrt(v1 + eps)
    r1 = [jnp.maximum((h - m1[:, None]) * s1[:, None] + be1_ref[...][:, None],
                      0.0) for h in h1]

    w2 = w2_ref[...]
    h2 = [jnp.dot(w2, r) + b2_ref[...][:, None] for r in r1]
    m2 = sum(jnp.sum(h, axis=1) for h in h2) * nrm
    v2 = sum(jnp.sum((h - m2[:, None]) ** 2, axis=1) for h in h2) * nrm
    s2 = g2_ref[...] / jnp.sqrt(v2 + eps)
    for b in range(B):
        o_ref[b] = jnp.maximum(
            (h2[b] - m2[:, None]) * s2[:, None] + be2_ref[...][:, None], 0.0)


def kernel(x, group_idx, fps_idx, N, W1, b1, gamma1, beta1, W2, b2, gamma2,
           beta2):
    xt = jnp.transpose(x, (0, 1, 3, 2))
    gif = jnp.transpose(group_idx, (0, 2, 1))

    dil_x, gx = _sc_scatter_gather(xt, gif, fps_idx)

    out = pl.pallas_call(
        _mlp_kernel,
        out_shape=jax.ShapeDtypeStruct((B, C, NPTS), jnp.float32),
    )(gx, dil_x, W1[:, :C], W1[:, C:], b1, gamma1, beta1, W2, b2, gamma2,
      beta2)
    return out
